# baseline (speedup 1.0000x reference)
import jax
import jax.numpy as jnp
from jax.experimental import pallas as pl
from jax.experimental.pallas import tpu as pltpu

_BM = 1024
_BK = 512


def _fused_kernel(x1_ref, x2_ref, w_ref, l1_ref, l2_ref, b_ref,
                  o1_ref, o2_ref, s1_ref, s2_ref):
    k = pl.program_id(1)

    @pl.when((pl.program_id(0) == 0) & (k == 0))
    def _init_supports():
        w = w_ref[...]
        s1_ref[...] = jnp.dot(x1_ref[...], w, preferred_element_type=jnp.float32)
        s2_ref[...] = jnp.dot(x2_ref[...], w, preferred_element_type=jnp.float32)

    s1 = s1_ref[pl.ds(k * _BK, _BK), :]
    s2 = s2_ref[pl.ds(k * _BK, _BK), :]
    p1 = jnp.dot(l1_ref[...], s1, preferred_element_type=jnp.float32)
    p2 = jnp.dot(l2_ref[...], s2, preferred_element_type=jnp.float32)

    @pl.when(k == 0)
    def _first():
        b = b_ref[...]
        o1_ref[...] = p1 + b
        o2_ref[...] = p2 + b

    @pl.when(k != 0)
    def _acc():
        o1_ref[...] += p1
        o2_ref[...] += p2


def kernel(node_input, hyperedge_input, node_lap, hyperedge_lap, weight, bias):
    n, f_in = node_input.shape
    m = hyperedge_input.shape[0]
    f_out = weight.shape[1]

    bias2d = bias.reshape(1, f_out)
    o1, o2 = pl.pallas_call(
        _fused_kernel,
        grid=(n // _BM, n // _BK),
        in_specs=[
            pl.BlockSpec((n, f_in), lambda i, k: (0, 0)),
            pl.BlockSpec((m, f_in), lambda i, k: (0, 0)),
            pl.BlockSpec((f_in, f_out), lambda i, k: (0, 0)),
            pl.BlockSpec((_BM, _BK), lambda i, k: (i, k)),
            pl.BlockSpec((_BM, _BK), lambda i, k: (i, k)),
            pl.BlockSpec((1, f_out), lambda i, k: (0, 0)),
        ],
        out_specs=(
            pl.BlockSpec((_BM, f_out), lambda i, k: (i, 0)),
            pl.BlockSpec((_BM, f_out), lambda i, k: (i, 0)),
        ),
        out_shape=(
            jax.ShapeDtypeStruct((n, f_out), jnp.float32),
            jax.ShapeDtypeStruct((m, f_out), jnp.float32),
        ),
        scratch_shapes=[
            pltpu.VMEM((n, f_out), jnp.float32),
            pltpu.VMEM((m, f_out), jnp.float32),
        ],
        compiler_params=pltpu.CompilerParams(
            dimension_semantics=("arbitrary", "arbitrary"),
        ),
    )(node_input, hyperedge_input, weight, node_lap, hyperedge_lap, bias2d)
    return o1, o2


# speedup vs baseline: 1.1651x; 1.1651x over previous
import jax
import jax.numpy as jnp
from jax.experimental import pallas as pl
from jax.experimental.pallas import tpu as pltpu

_BLOCK_ROWS = 256


def _fused_kernel(x1_ref, x2_ref, w_ref, l1_ref, l2_ref, b_ref,
                  o1_ref, o2_ref, s1_ref, s2_ref):
    @pl.when(pl.program_id(0) == 0)
    def _init():
        w = w_ref[...]
        s1_ref[...] = jnp.dot(x1_ref[...], w,
                              preferred_element_type=jnp.float32
                              ).astype(jnp.bfloat16)
        s2_ref[...] = jnp.dot(x2_ref[...], w,
                              preferred_element_type=jnp.float32
                              ).astype(jnp.bfloat16)

    b = b_ref[...]
    l1 = l1_ref[...].astype(jnp.bfloat16)
    l2 = l2_ref[...].astype(jnp.bfloat16)
    o1_ref[...] = jnp.dot(l1, s1_ref[...],
                          preferred_element_type=jnp.float32) + b
    o2_ref[...] = jnp.dot(l2, s2_ref[...],
                          preferred_element_type=jnp.float32) + b


def kernel(node_input, hyperedge_input, node_lap, hyperedge_lap, weight, bias):
    n, f_in = node_input.shape
    m = hyperedge_input.shape[0]
    f_out = weight.shape[1]

    bias2d = bias.reshape(1, f_out)
    blk = _BLOCK_ROWS
    o1, o2 = pl.pallas_call(
        _fused_kernel,
        grid=(n // blk,),
        in_specs=[
            pl.BlockSpec((n, f_in), lambda i: (0, 0)),
            pl.BlockSpec((m, f_in), lambda i: (0, 0)),
            pl.BlockSpec((f_in, f_out), lambda i: (0, 0)),
            pl.BlockSpec((blk, n), lambda i: (i, 0)),
            pl.BlockSpec((blk, m), lambda i: (i, 0)),
            pl.BlockSpec((1, f_out), lambda i: (0, 0)),
        ],
        out_specs=(
            pl.BlockSpec((blk, f_out), lambda i: (i, 0)),
            pl.BlockSpec((blk, f_out), lambda i: (i, 0)),
        ),
        out_shape=(
            jax.ShapeDtypeStruct((n, f_out), jnp.float32),
            jax.ShapeDtypeStruct((m, f_out), jnp.float32),
        ),
        scratch_shapes=[
            pltpu.VMEM((n, f_out), jnp.bfloat16),
            pltpu.VMEM((m, f_out), jnp.bfloat16),
        ],
        compiler_params=pltpu.CompilerParams(
            dimension_semantics=("arbitrary",),
        ),
    )(node_input, hyperedge_input, weight, node_lap, hyperedge_lap, bias2d)
    return o1, o2
